# baseline (speedup 1.0000x reference)
import functools

import jax
import jax.numpy as jnp
from jax import lax
from jax.experimental import pallas as pl
from jax.experimental.pallas import tpu as pltpu
from jax.experimental.pallas import tpu_sc as plsc

B, C, N = 64, 96, 576
NPOINT = 288
NBLK = N // 16
NW = 32


def _fps_body(x_hbm, x1_hbm, idx_hbm, x_v, x1_v, dist_v, idx_v, act_v):
    nc = 2
    wid = lax.axis_index("s") * nc + lax.axis_index("c")
    lanes = lax.broadcasted_iota(jnp.int32, (16,), 0)
    lane0 = lanes == 0
    G = 3

    for r in range(B // NW):
        b = wid + NW * r
        pltpu.sync_copy(x_hbm.at[b], x_v)

        big = jnp.full((16,), 1e10, jnp.float32)
        def init_j(j):
            dist_v[pl.ds(j * 16, 16)] = big
            act_v[pl.ds(j * 16, 16)] = j * 16 + lanes
        pl.loop(0, NBLK)(init_j)

        def iter_body(i, carry):
            far, pos, nact, acc = carry
            acc = jnp.where(lanes == (i & 15), jnp.broadcast_to(far, (16,)), acc)

            @pl.when((i & 15) == 15)
            def _flush():
                idx_v[pl.ds(i - 15, 16)] = acc

            vpos = jnp.broadcast_to(pos, (16,))
            cvs0 = [plsc.load_gather(x_v, [cb * 16 + lanes, vpos])
                    for cb in range(6)]
            for cb in range(6):
                x1_v[i, pl.ds(cb * 16, 16)] = cvs0[cb]

            last = nact - 1
            vlast = jnp.broadcast_to(last, (16,))
            for cb in range(6):
                lastcol = plsc.load_gather(x_v, [cb * 16 + lanes, vlast])
                plsc.store_scatter(x_v, [cb * 16 + lanes, vpos], lastcol)
            last_id = plsc.load_gather(act_v, [vlast])
            last_d = plsc.load_gather(dist_v, [vlast])
            plsc.store_scatter(act_v, [vpos], last_id, mask=lane0)
            plsc.store_scatter(dist_v, [vpos], last_d, mask=lane0)
            plsc.store_scatter(dist_v, [vlast],
                               jnp.full((16,), -1.0, jnp.float32), mask=lane0)

            nblk = (last + 15) >> 4
            nchunk = ((nblk + 2) * 86) >> 8

            def chunk(ci, carry2):
                bval, borig, bpos = carry2[0], carry2[1], carry2[2]
                cvs = carry2[3:]
                j0 = ci * G
                p15 = [None] * G
                p1537 = [None] * G
                p26 = [None] * G
                hold = [None] * G
                dnew = [None] * G
                for s in (1, 5, 3, 7, 2, 6, 0, 4):
                    t = [None] * G
                    for k in range(12):
                        c = 8 * k + s
                        bc = jnp.broadcast_to(cvs[c // 16][c % 16], (16,))
                        for b in range(G):
                            xv = x_v[c, pl.ds((j0 + b) * 16, 16)]
                            d = xv - bc
                            sq = d * d
                            t[b] = sq if k == 0 else t[b] + sq
                    for b in range(G):
                        if s == 1:
                            hold[b] = t[b]
                        elif s == 5:
                            p15[b] = hold[b] + t[b]
                        elif s == 3:
                            hold[b] = t[b]
                        elif s == 7:
                            p1537[b] = p15[b] + (hold[b] + t[b])
                        elif s == 2:
                            hold[b] = t[b]
                        elif s == 6:
                            p26[b] = hold[b] + t[b]
                        elif s == 0:
                            hold[b] = t[b]
                        else:
                            dnew[b] = p1537[b] + (p26[b] + (hold[b] + t[b]))
                for b in range(G):
                    sl = pl.ds((j0 + b) * 16, 16)
                    dold = dist_v[sl]
                    dmin = jnp.minimum(dold, dnew[b])
                    dist_v[sl] = dmin
                    ids_b = act_v[sl]
                    posv = (j0 + b) * 16 + lanes
                    better = ((dmin > bval)
                              | ((dmin == bval) & (ids_b < borig)))
                    bval = jnp.where(better, dmin, bval)
                    borig = jnp.where(better, ids_b, borig)
                    bpos = jnp.where(better, posv, bpos)
                zero16 = jnp.zeros((16,), jnp.float32)
                cvs = tuple(jnp.where(far < 0, zero16, cv) for cv in cvs)
                return (bval, borig, bpos) + cvs

            bval0 = jnp.full((16,), -1.0, jnp.float32)
            borig0 = jnp.zeros((16,), jnp.int32)
            bpos0 = jnp.zeros((16,), jnp.int32)
            out = plsc.parallel_loop(
                0, nchunk, unroll=3,
                carry=(bval0, borig0, bpos0) + tuple(cvs0))(chunk)
            bval, borig, bpos = out[0], out[1], out[2]
            m = jnp.max(bval)
            ism = bval == m
            morig = jnp.min(jnp.where(ism, borig, N))
            mpos = jnp.min(jnp.where(ism & (borig == morig), bpos, N))
            return (morig.astype(jnp.int32), mpos.astype(jnp.int32),
                    last, acc)

        far0 = jnp.zeros((), jnp.int32)
        pos0 = jnp.zeros((), jnp.int32)
        nact0 = jnp.full((), N, jnp.int32)
        acc0 = jnp.zeros((16,), jnp.int32)
        lax.fori_loop(0, NPOINT, iter_body, (far0, pos0, nact0, acc0))

        pltpu.sync_copy(idx_v, idx_hbm.at[b])
        pltpu.sync_copy(x1_v, x1_hbm.at[b])


_fps = functools.partial(
    pl.kernel,
    mesh=plsc.VectorSubcoreMesh(core_axis_name="c", subcore_axis_name="s"),
    compiler_params=pltpu.CompilerParams(use_tc_tiling_on_sc=False,
                                         needs_layout_passes=False),
    out_type=[
        jax.ShapeDtypeStruct((B, NPOINT, C), jnp.float32),
        jax.ShapeDtypeStruct((B, NPOINT), jnp.int32),
    ],
    scratch_types=[
        pltpu.VMEM((C, N + 1), jnp.float32),
        pltpu.VMEM((NPOINT, C), jnp.float32),
        pltpu.VMEM((N,), jnp.float32),
        pltpu.VMEM((NPOINT,), jnp.int32),
        pltpu.VMEM((N,), jnp.int32),
    ],
)(_fps_body)


def _tr_body(x_ref, o_ref):
    o_ref[...] = jnp.transpose(x_ref[...], (0, 2, 1))


def _transpose_x(x):
    return pl.pallas_call(
        _tr_body,
        grid=(8,),
        in_specs=[pl.BlockSpec((8, C, N), lambda i: (i, 0, 0))],
        out_specs=pl.BlockSpec((8, N, C), lambda i: (i, 0, 0)),
        out_shape=jax.ShapeDtypeStruct((B, N, C), jnp.float32),
    )(x)


def kernel(x):
    x0 = _transpose_x(x)
    xp = jnp.pad(x, ((0, 0), (0, 0), (0, 1)))
    x1, idxs = _fps(xp)
    return (x0, x1, idxs)


# speedup vs baseline: 1.2125x; 1.2125x over previous
import functools

import jax
import jax.numpy as jnp
from jax import lax
from jax.experimental import pallas as pl
from jax.experimental.pallas import tpu as pltpu
from jax.experimental.pallas import tpu_sc as plsc

B, C, N = 64, 96, 576
NPOINT = 288
NBLK = N // 16
NW = 32


def _fps_body(x_hbm, x1_hbm, idx_hbm, x_v, x1_v, dist_v, idx_v, act_v):
    nc = 2
    wid = lax.axis_index("s") * nc + lax.axis_index("c")
    lanes = lax.broadcasted_iota(jnp.int32, (16,), 0)
    lane0 = lanes == 0
    G = 3

    for r in range(B // NW):
        b = wid + NW * r
        pltpu.sync_copy(x_hbm.at[b], x_v)

        big = jnp.full((16,), 1e10, jnp.float32)
        def init_j(j):
            dist_v[pl.ds(j * 16, 16)] = big
            act_v[pl.ds(j * 16, 16)] = j * 16 + lanes
        pl.loop(0, NBLK)(init_j)

        def iter_body(i, carry):
            far, pos, nact, acc = carry
            acc = jnp.where(lanes == (i & 15), jnp.broadcast_to(far, (16,)), acc)

            @pl.when((i & 15) == 15)
            def _flush():
                idx_v[pl.ds(i - 15, 16)] = acc

            vpos = jnp.broadcast_to(pos, (16,))
            cvs0 = [plsc.load_gather(x_v, [cb * 16 + lanes, vpos])
                    for cb in range(6)]
            for cb in range(6):
                x1_v[i, pl.ds(cb * 16, 16)] = cvs0[cb]

            last = nact - 1
            vlast = jnp.broadcast_to(last, (16,))
            for cb in range(6):
                lastcol = plsc.load_gather(x_v, [cb * 16 + lanes, vlast])
                plsc.store_scatter(x_v, [cb * 16 + lanes, vpos], lastcol)
            last_id = plsc.load_gather(act_v, [vlast])
            last_d = plsc.load_gather(dist_v, [vlast])
            plsc.store_scatter(act_v, [vpos], last_id, mask=lane0)
            plsc.store_scatter(dist_v, [vpos], last_d, mask=lane0)
            plsc.store_scatter(dist_v, [vlast],
                               jnp.full((16,), -1.0, jnp.float32), mask=lane0)

            nblk = (last + 15) >> 4
            nchunk = ((nblk + 2) * 86) >> 8

            def chunk(ci, carry2):
                bval, borig, bpos = carry2[0], carry2[1], carry2[2]
                cvs = carry2[3:]
                j0 = ci * G
                p15 = [None] * G
                p1537 = [None] * G
                p26 = [None] * G
                hold = [None] * G
                dnew = [None] * G
                for s in (1, 5, 3, 7, 2, 6, 0, 4):
                    t = [None] * G
                    for k in range(12):
                        c = 8 * k + s
                        bc = jnp.broadcast_to(cvs[c // 16][c % 16], (16,))
                        for b in range(G):
                            xv = x_v[c, pl.ds((j0 + b) * 16, 16)]
                            d = xv - bc
                            sq = d * d
                            t[b] = sq if k == 0 else t[b] + sq
                    for b in range(G):
                        if s == 1:
                            hold[b] = t[b]
                        elif s == 5:
                            p15[b] = hold[b] + t[b]
                        elif s == 3:
                            hold[b] = t[b]
                        elif s == 7:
                            p1537[b] = p15[b] + (hold[b] + t[b])
                        elif s == 2:
                            hold[b] = t[b]
                        elif s == 6:
                            p26[b] = hold[b] + t[b]
                        elif s == 0:
                            hold[b] = t[b]
                        else:
                            dnew[b] = p1537[b] + (p26[b] + (hold[b] + t[b]))
                for b in range(G):
                    sl = pl.ds((j0 + b) * 16, 16)
                    dold = dist_v[sl]
                    dmin = jnp.minimum(dold, dnew[b])
                    dist_v[sl] = dmin
                    ids_b = act_v[sl]
                    posv = (j0 + b) * 16 + lanes
                    better = ((dmin > bval)
                              | ((dmin == bval) & (ids_b < borig)))
                    bval = jnp.where(better, dmin, bval)
                    borig = jnp.where(better, ids_b, borig)
                    bpos = jnp.where(better, posv, bpos)
                zero16 = jnp.zeros((16,), jnp.float32)
                cvs = tuple(jnp.where(far < 0, zero16, cv) for cv in cvs)
                return (bval, borig, bpos) + cvs

            bval0 = jnp.full((16,), -1.0, jnp.float32)
            borig0 = jnp.zeros((16,), jnp.int32)
            bpos0 = jnp.zeros((16,), jnp.int32)
            out = plsc.parallel_loop(
                0, nchunk, unroll=2,
                carry=(bval0, borig0, bpos0) + tuple(cvs0))(chunk)
            bval, borig, bpos = out[0], out[1], out[2]
            m = jnp.max(bval)
            ism = bval == m
            morig = jnp.min(jnp.where(ism, borig, N))
            mpos = jnp.min(jnp.where(ism & (borig == morig), bpos, N))
            return (morig.astype(jnp.int32), mpos.astype(jnp.int32),
                    last, acc)

        far0 = jnp.zeros((), jnp.int32)
        pos0 = jnp.zeros((), jnp.int32)
        nact0 = jnp.full((), N, jnp.int32)
        acc0 = jnp.zeros((16,), jnp.int32)
        lax.fori_loop(0, NPOINT, iter_body, (far0, pos0, nact0, acc0))

        pltpu.sync_copy(idx_v, idx_hbm.at[b])
        pltpu.sync_copy(x1_v, x1_hbm.at[b])


_fps = functools.partial(
    pl.kernel,
    mesh=plsc.VectorSubcoreMesh(core_axis_name="c", subcore_axis_name="s"),
    compiler_params=pltpu.CompilerParams(use_tc_tiling_on_sc=False,
                                         needs_layout_passes=False),
    out_type=[
        jax.ShapeDtypeStruct((B, NPOINT, C), jnp.float32),
        jax.ShapeDtypeStruct((B, NPOINT), jnp.int32),
    ],
    scratch_types=[
        pltpu.VMEM((C, N + 1), jnp.float32),
        pltpu.VMEM((NPOINT, C), jnp.float32),
        pltpu.VMEM((N,), jnp.float32),
        pltpu.VMEM((NPOINT,), jnp.int32),
        pltpu.VMEM((N,), jnp.int32),
    ],
)(_fps_body)


def _tr_body(x_ref, o_ref):
    o_ref[...] = jnp.transpose(x_ref[...], (0, 2, 1))


def _transpose_x(x):
    return pl.pallas_call(
        _tr_body,
        grid=(8,),
        in_specs=[pl.BlockSpec((8, C, N), lambda i: (i, 0, 0))],
        out_specs=pl.BlockSpec((8, N, C), lambda i: (i, 0, 0)),
        out_shape=jax.ShapeDtypeStruct((B, N, C), jnp.float32),
    )(x)


def kernel(x):
    x0 = _transpose_x(x)
    xp = jnp.pad(x, ((0, 0), (0, 0), (0, 1)))
    x1, idxs = _fps(xp)
    return (x0, x1, idxs)


# speedup vs baseline: 1.2127x; 1.0002x over previous
import functools

import jax
import jax.numpy as jnp
from jax import lax
from jax.experimental import pallas as pl
from jax.experimental.pallas import tpu as pltpu
from jax.experimental.pallas import tpu_sc as plsc

B, C, N = 64, 96, 576
NPOINT = 288
NBLK = N // 16
NW = 32


def _fps_body(x_hbm, x1_hbm, idx_hbm, x_v, x1_v, dist_v, idx_v, act_v):
    nc = 2
    wid = lax.axis_index("s") * nc + lax.axis_index("c")
    lanes = lax.broadcasted_iota(jnp.int32, (16,), 0)
    lane0 = lanes == 0
    G = 3

    for r in range(B // NW):
        b = wid + NW * r
        pltpu.sync_copy(x_hbm.at[b], x_v)

        big = jnp.full((16,), 1e10, jnp.float32)
        def init_j(j):
            dist_v[pl.ds(j * 16, 16)] = big
            act_v[pl.ds(j * 16, 16)] = j * 16 + lanes
        pl.loop(0, NBLK)(init_j)

        def iter_body(i, carry):
            far, pos, nact, acc = carry
            acc = jnp.where(lanes == (i & 15), jnp.broadcast_to(far, (16,)), acc)

            @pl.when((i & 15) == 15)
            def _flush():
                idx_v[pl.ds(i - 15, 16)] = acc

            vpos = jnp.broadcast_to(pos, (16,))
            cvs0 = [plsc.load_gather(x_v, [cb * 16 + lanes, vpos])
                    for cb in range(6)]
            for cb in range(6):
                x1_v[i, pl.ds(cb * 16, 16)] = cvs0[cb]

            last = nact - 1
            vlast = jnp.broadcast_to(last, (16,))
            for cb in range(6):
                lastcol = plsc.load_gather(x_v, [cb * 16 + lanes, vlast])
                plsc.store_scatter(x_v, [cb * 16 + lanes, vpos], lastcol)
            last_id = plsc.load_gather(act_v, [vlast])
            last_d = plsc.load_gather(dist_v, [vlast])
            plsc.store_scatter(act_v, [vpos], last_id, mask=lane0)
            plsc.store_scatter(dist_v, [vpos], last_d, mask=lane0)
            plsc.store_scatter(dist_v, [vlast],
                               jnp.full((16,), -1.0, jnp.float32), mask=lane0)

            nblk = (last + 15) >> 4
            nchunk = ((nblk + 2) * 86) >> 8

            def chunk(ci, carry2):
                bval, borig, bpos = carry2[0], carry2[1], carry2[2]
                cvs = carry2[3:]
                j0 = ci * G
                p15 = [None] * G
                p1537 = [None] * G
                p26 = [None] * G
                hold = [None] * G
                dnew = [None] * G
                for s in (1, 5, 3, 7, 2, 6, 0, 4):
                    t = [None] * G
                    for k in range(12):
                        c = 8 * k + s
                        bc = jnp.broadcast_to(cvs[c // 16][c % 16], (16,))
                        for b in range(G):
                            xv = x_v[c, pl.ds((j0 + b) * 16, 16)]
                            d = xv - bc
                            sq = d * d
                            t[b] = sq if k == 0 else t[b] + sq
                    for b in range(G):
                        if s == 1:
                            hold[b] = t[b]
                        elif s == 5:
                            p15[b] = hold[b] + t[b]
                        elif s == 3:
                            hold[b] = t[b]
                        elif s == 7:
                            p1537[b] = p15[b] + (hold[b] + t[b])
                        elif s == 2:
                            hold[b] = t[b]
                        elif s == 6:
                            p26[b] = hold[b] + t[b]
                        elif s == 0:
                            hold[b] = t[b]
                        else:
                            dnew[b] = p1537[b] + (p26[b] + (hold[b] + t[b]))
                for b in range(G):
                    sl = pl.ds((j0 + b) * 16, 16)
                    dold = dist_v[sl]
                    dmin = jnp.minimum(dold, dnew[b])
                    dist_v[sl] = dmin
                    ids_b = act_v[sl]
                    posv = (j0 + b) * 16 + lanes
                    better = ((dmin > bval)
                              | ((dmin == bval) & (ids_b < borig)))
                    bval = jnp.where(better, dmin, bval)
                    borig = jnp.where(better, ids_b, borig)
                    bpos = jnp.where(better, posv, bpos)
                zero16 = jnp.zeros((16,), jnp.float32)
                cvs = tuple(jnp.where(far < 0, zero16, cv) for cv in cvs)
                return (bval, borig, bpos) + cvs

            bval0 = jnp.full((16,), -1.0, jnp.float32)
            borig0 = jnp.zeros((16,), jnp.int32)
            bpos0 = jnp.zeros((16,), jnp.int32)
            out = plsc.parallel_loop(
                0, nchunk, unroll=2,
                carry=(bval0, borig0, bpos0) + tuple(cvs0))(chunk)
            bval, borig, bpos = out[0], out[1], out[2]
            m = jnp.max(bval)
            ism = bval == m
            morig = jnp.min(jnp.where(ism, borig, N))
            mpos = jnp.min(jnp.where(ism & (borig == morig), bpos, N))
            return (morig.astype(jnp.int32), mpos.astype(jnp.int32),
                    last, acc)

        far0 = jnp.zeros((), jnp.int32)
        pos0 = jnp.zeros((), jnp.int32)
        nact0 = jnp.full((), N, jnp.int32)
        acc0 = jnp.zeros((16,), jnp.int32)
        lax.fori_loop(0, NPOINT, iter_body, (far0, pos0, nact0, acc0))

        pltpu.sync_copy(idx_v, idx_hbm.at[b])
        pltpu.sync_copy(x1_v, x1_hbm.at[b])


def _make_fps():
    return functools.partial(
        pl.kernel,
        mesh=plsc.VectorSubcoreMesh(core_axis_name="c", subcore_axis_name="s"),
        compiler_params=pltpu.CompilerParams(use_tc_tiling_on_sc=False,
                                             needs_layout_passes=False),
        out_type=[
            jax.ShapeDtypeStruct((B, NPOINT, C), jnp.float32),
            jax.ShapeDtypeStruct((B, NPOINT), jnp.int32),
        ],
        scratch_types=[
            pltpu.VMEM((C, N + 1), jnp.float32),
            pltpu.VMEM((NPOINT, C), jnp.float32),
            pltpu.VMEM((N,), jnp.float32),
            pltpu.VMEM((NPOINT,), jnp.int32),
            pltpu.VMEM((N,), jnp.int32),
        ],
    )(_fps_body)


def _tr_body(x_ref, o_ref):
    o_ref[...] = jnp.transpose(x_ref[...], (0, 2, 1))


def _transpose_x(x):
    return pl.pallas_call(
        _tr_body,
        grid=(8,),
        in_specs=[pl.BlockSpec((8, C, N), lambda i: (i, 0, 0))],
        out_specs=pl.BlockSpec((8, N, C), lambda i: (i, 0, 0)),
        out_shape=jax.ShapeDtypeStruct((B, N, C), jnp.float32),
    )(x)


def kernel(x):
    x0 = _transpose_x(x)
    xp = jnp.pad(x, ((0, 0), (0, 0), (0, 1)))
    x1, idxs = _make_fps()(xp)
    return (x0, x1, idxs)
